# Initial kernel scaffold; baseline (speedup 1.0000x reference)
import jax
import jax.numpy as jnp
from jax.experimental import pallas as pl


def kernel(weights, ray_indices, num_rays):
    raise NotImplementedError("write your pallas kernel here")


# speedup vs baseline: 26.7503x; 26.7503x over previous
import functools

import jax
import jax.numpy as jnp
from jax import lax
from jax.experimental import pallas as pl
from jax.experimental.pallas import tpu as pltpu
from jax.experimental.pallas import tpu_sc as plsc

_N_RAYS = 100000
_NR_PAD = 102400
_PER_TILE_OUT = _NR_PAD // 16

_NC, _NS = 2, 16
_NW = _NC * _NS

_LANE = 128
_W = 40
_WINS = 39
_ROWS_PER_TILE = _W * _WINS


def _sc_partial_sums(idx2d, w2d):
    n_rows = idx2d.shape[0]
    extra_rows = n_rows - _NW * _ROWS_PER_TILE

    mesh = plsc.VectorSubcoreMesh(core_axis_name="c", subcore_axis_name="s")

    @functools.partial(
        pl.kernel,
        out_type=jax.ShapeDtypeStruct((_NC, _NR_PAD), jnp.float32),
        mesh=mesh,
        scratch_types=[
            pltpu.VMEM((_W, _LANE), jnp.int32),
            pltpu.VMEM((_W, _LANE), jnp.float32),
            pltpu.VMEM((_PER_TILE_OUT,), jnp.float32),
            pltpu.VMEM_SHARED((_NR_PAD,), jnp.float32),
        ],
    )
    def k(idx_hbm, w_hbm, out_hbm, iv, wv, zb, acc):
        c = lax.axis_index("c")
        s = lax.axis_index("s")
        tid = c * _NS + s

        def zloop(i, _):
            zb[pl.ds(i * 16, 16)] = jnp.zeros((16,), jnp.float32)
            return 0

        lax.fori_loop(0, _PER_TILE_OUT // 16, zloop, 0)
        pltpu.sync_copy(zb, acc.at[pl.ds(s * _PER_TILE_OUT, _PER_TILE_OUT)])
        plsc.subcore_barrier()

        def scatter_row(j, _):
            pltpu.sync_copy(wv.at[j], acc.at[iv.at[j]], add=True)
            return 0

        row0 = tid * _ROWS_PER_TILE

        def window(g, _):
            base = row0 + g * _W
            pltpu.sync_copy(idx_hbm.at[pl.ds(base, _W)], iv)
            pltpu.sync_copy(w_hbm.at[pl.ds(base, _W)], wv)
            lax.fori_loop(0, _W, scatter_row, 0)
            return 0

        lax.fori_loop(0, _WINS, window, 0)

        if extra_rows > 0:
            assert extra_rows % _W == 0
            @pl.when(tid == _NW - 1)
            def _():
                def rwindow(g, _):
                    base = _NW * _ROWS_PER_TILE + g * _W
                    pltpu.sync_copy(idx_hbm.at[pl.ds(base, _W)], iv)
                    pltpu.sync_copy(w_hbm.at[pl.ds(base, _W)], wv)
                    lax.fori_loop(0, _W, scatter_row, 0)
                    return 0

                lax.fori_loop(0, extra_rows // _W, rwindow, 0)

        plsc.subcore_barrier()
        pltpu.sync_copy(acc.at[pl.ds(s * _PER_TILE_OUT, _PER_TILE_OUT)],
                        out_hbm.at[c, pl.ds(s * _PER_TILE_OUT, _PER_TILE_OUT)])

    return k(idx2d, w2d)


def _combine_body(p_ref, o_ref):
    o_ref[...] = jnp.clip(p_ref[0] + p_ref[1], 0.0, 1.0)


def kernel(weights, ray_indices, num_rays):
    del num_rays
    idx = ray_indices.astype(jnp.int32)
    w = weights.astype(jnp.float32)
    n = w.shape[0]
    assert n % _LANE == 0
    idx2d = idx.reshape(n // _LANE, _LANE)
    w2d = w.reshape(n // _LANE, _LANE)

    partial = _sc_partial_sums(idx2d, w2d)

    p3 = partial.reshape(_NC, _NR_PAD // 128, 128)
    out = pl.pallas_call(
        _combine_body,
        out_shape=jax.ShapeDtypeStruct((_NR_PAD // 128, 128), jnp.float32),
    )(p3)
    return out.reshape(_NR_PAD)[:_N_RAYS][:, None]


# speedup vs baseline: 40.1829x; 1.5021x over previous
import functools

import jax
import jax.numpy as jnp
from jax import lax
from jax.experimental import pallas as pl
from jax.experimental.pallas import tpu as pltpu
from jax.experimental.pallas import tpu_sc as plsc

_N_RAYS = 100000

_NC, _NS = 2, 16
_NW = _NC * _NS

_WC = 284
_CH = 16 * _WC
_WINS = 44
_TILE_SAMP = _WINS * _CH
_UNROLL = 4

_ACC = 102656
_DUMMY = _ACC - 1


def _sc_partial_sums(idx1d, w1d):
    n = idx1d.shape[0]
    extra = n - _NW * _TILE_SAMP
    assert extra % 16 == 0
    wce = extra // 16
    assert wce % _UNROLL == 0 and _WC % _UNROLL == 0

    mesh = plsc.VectorSubcoreMesh(core_axis_name="c", subcore_axis_name="s")

    @functools.partial(
        pl.kernel,
        out_type=jax.ShapeDtypeStruct((_NW, _ACC), jnp.float32),
        mesh=mesh,
        compiler_params=pltpu.CompilerParams(needs_layout_passes=False),
        scratch_types=[
            pltpu.VMEM((2 * _CH,), jnp.int32),
            pltpu.VMEM((2 * _CH,), jnp.float32),
            pltpu.VMEM((_ACC,), jnp.float32),
            pltpu.SemaphoreType.DMA((2,)),
        ],
    )
    def k(idx_hbm, w_hbm, out_hbm, ibuf, wbuf, acc, sem):
        c = lax.axis_index("c")
        s = lax.axis_index("s")
        tid = c * _NS + s
        tbase = tid * _TILE_SAMP

        lane_id = lax.iota(jnp.int32, 16)
        zeros16 = jnp.zeros((16,), jnp.float32)
        dummy16 = jnp.full((16,), _DUMMY, jnp.int32)

        def zloop(i, _):
            acc[pl.ds(i * 16, 16)] = zeros16
            return 0

        lax.fori_loop(0, _ACC // 16, zloop, 0)

        def issue(w, p):
            base = tbase + w * _CH
            pltpu.make_async_copy(idx_hbm.at[pl.ds(base, _CH)],
                                  ibuf.at[pl.ds(p * _CH, _CH)],
                                  sem.at[p]).start()
            pltpu.make_async_copy(w_hbm.at[pl.ds(base, _CH)],
                                  wbuf.at[pl.ds(p * _CH, _CH)],
                                  sem.at[p]).start()

        def wait_win(w, p):
            base = tbase + w * _CH
            pltpu.make_async_copy(idx_hbm.at[pl.ds(base, _CH)],
                                  ibuf.at[pl.ds(p * _CH, _CH)],
                                  sem.at[p]).wait()
            pltpu.make_async_copy(w_hbm.at[pl.ds(base, _CH)],
                                  wbuf.at[pl.ds(p * _CH, _CH)],
                                  sem.at[p]).wait()

        def run_window(p, wc):
            lane_off = lane_id * wc + p * _CH

            def step_block(i, carry):
                cur, accv = carry
                t = i * _UNROLL
                for u in range(_UNROLL):
                    ivec = lane_off + (t + u)
                    idxv = plsc.load_gather(ibuf, [ivec])
                    wvv = plsc.load_gather(wbuf, [ivec])
                    m = idxv != cur
                    plsc.addupdate_scatter(acc, [cur], accv, mask=m)
                    accv = jnp.where(m, wvv, accv + wvv)
                    cur = idxv
                return cur, accv

            cur, accv = lax.fori_loop(0, wc // _UNROLL, step_block,
                                      (dummy16, zeros16))
            for i in range(16):
                plsc.addupdate_scatter(acc, [cur], accv, mask=lane_id == i)

        issue(0, 0)

        def wloop(w, _):
            p = w % 2
            wait_win(w, p)

            @pl.when(w + 1 < _WINS)
            def _():
                issue(w + 1, (w + 1) % 2)

            run_window(p, _WC)
            return 0

        lax.fori_loop(0, _WINS, wloop, 0)

        if extra > 0:
            @pl.when(tid == _NW - 1)
            def _():
                base = _NW * _TILE_SAMP
                pltpu.sync_copy(idx_hbm.at[pl.ds(base, extra)],
                                ibuf.at[pl.ds(0, extra)])
                pltpu.sync_copy(w_hbm.at[pl.ds(base, extra)],
                                wbuf.at[pl.ds(0, extra)])
                run_window(0, wce)

        pltpu.sync_copy(acc, out_hbm.at[tid])

    return k(idx1d, w1d)


def _combine_body(p_ref, o_ref):
    o_ref[...] = jnp.clip(jnp.sum(p_ref[...], axis=0), 0.0, 1.0)


def kernel(weights, ray_indices, num_rays):
    del num_rays
    idx = ray_indices.astype(jnp.int32)
    w = weights.astype(jnp.float32)

    partial = _sc_partial_sums(idx, w)

    p3 = partial.reshape(_NW, _ACC // 128, 128)
    out = pl.pallas_call(
        _combine_body,
        out_shape=jax.ShapeDtypeStruct((_ACC // 128, 128), jnp.float32),
    )(p3)
    return out.reshape(_ACC)[:_N_RAYS][:, None]


# speedup vs baseline: 108.8237x; 2.7082x over previous
import functools

import jax
import jax.numpy as jnp
from jax import lax
from jax.experimental import pallas as pl
from jax.experimental.pallas import tpu as pltpu
from jax.experimental.pallas import tpu_sc as plsc

_N_RAYS = 100000

_NC, _NS = 2, 16
_NW = _NC * _NS

_WC = 125
_CH = 16 * _WC
_WINS = 100
_TILE_SAMP = _WINS * _CH
_UNROLL = 5

_ACC = 102656
_DUMMY = _ACC - 1
_SROWS = 896


def _sc_partial_sums(idx1d, w1d):
    assert idx1d.shape[0] == _NW * _TILE_SAMP
    assert _WC % _UNROLL == 0

    mesh = plsc.VectorSubcoreMesh(core_axis_name="c", subcore_axis_name="s")

    @functools.partial(
        pl.kernel,
        out_type=jax.ShapeDtypeStruct((_NC, _SROWS, 128), jnp.float32),
        mesh=mesh,
        compiler_params=pltpu.CompilerParams(needs_layout_passes=False),
        scratch_types=[
            pltpu.VMEM((3 * _CH,), jnp.int32),
            pltpu.VMEM((3 * _CH,), jnp.float32),
            pltpu.VMEM((_ACC // 128, 128), jnp.float32),
            pltpu.VMEM((16,), jnp.int32),
            pltpu.VMEM_SHARED((_SROWS, 128), jnp.float32),
            pltpu.SemaphoreType.DMA((3,)),
        ],
    )
    def k(idx_hbm, w_hbm, out_hbm, ibuf, wbuf, acc, tailbuf, spacc, sem):
        c = lax.axis_index("c")
        s = lax.axis_index("s")
        tid = c * _NS + s
        tbase = tid * _TILE_SAMP

        lane_id = lax.iota(jnp.int32, 16)
        zeros16 = jnp.zeros((16,), jnp.float32)
        dummy16 = jnp.full((16,), _DUMMY, jnp.int32)

        def issue(w, p):
            base = tbase + w * _CH
            pltpu.make_async_copy(idx_hbm.at[pl.ds(base, _CH)],
                                  ibuf.at[pl.ds(p * _CH, _CH)],
                                  sem.at[p]).start()
            pltpu.make_async_copy(w_hbm.at[pl.ds(base, _CH)],
                                  wbuf.at[pl.ds(p * _CH, _CH)],
                                  sem.at[p]).start()

        def wait_win(w, p):
            base = tbase + w * _CH
            pltpu.make_async_copy(idx_hbm.at[pl.ds(base, _CH)],
                                  ibuf.at[pl.ds(p * _CH, _CH)],
                                  sem.at[p]).wait()
            pltpu.make_async_copy(w_hbm.at[pl.ds(base, _CH)],
                                  wbuf.at[pl.ds(p * _CH, _CH)],
                                  sem.at[p]).wait()

        def run_window(p, wc):
            lane_off = lane_id * wc + p * _CH

            def step_block(i, carry):
                cur, accv, base = carry
                t = i * _UNROLL
                loaded = []
                for u in range(_UNROLL):
                    ivec = lane_off + (t + u)
                    loaded.append((plsc.load_gather(ibuf, [ivec]),
                                   plsc.load_gather(wbuf, [ivec])))
                for idxv, wvv in loaded:
                    m = idxv != cur
                    plsc.addupdate_scatter(
                        acc,
                        [lax.shift_right_logical(cur, 7),
                         lax.bitwise_and(cur, 127)],
                        accv - base, mask=m)
                    base = jnp.where(m, accv, base)
                    accv = accv + wvv
                    cur = idxv
                return cur, accv, base

            cur, accv, base = lax.fori_loop(0, wc // _UNROLL, step_block,
                                            (dummy16, zeros16, zeros16))
            row = lax.shift_right_logical(cur, 7)
            col = lax.bitwise_and(cur, 127)
            for i in range(16):
                plsc.addupdate_scatter(acc, [row, col], accv - base,
                                       mask=lane_id == i)

        issue(0, 0)
        issue(1, 1)
        pltpu.sync_copy(idx_hbm.at[pl.ds(tbase + _TILE_SAMP - 16, 16)],
                        tailbuf)
        wait_win(0, 0)
        lo = ibuf[pl.ds(0, 16)][0]
        hi = tailbuf[...][15]
        r0 = lax.shift_right_logical(lo, 7)
        r1 = lax.shift_right_logical(hi, 7)
        cnt = lax.shift_right_logical(r1 - r0, 4) + 1

        def zchunk(k_, _):
            rb = r0 + k_ * 16
            for r in range(16):
                for u in range(8):
                    acc[rb + r, pl.ds(u * 16, 16)] = zeros16
            return 0

        lax.fori_loop(0, cnt, zchunk, 0)

        for k_ in range(4):
            pltpu.sync_copy(acc.at[pl.ds(r0, 14)],
                            spacc.at[pl.ds(s * 56 + k_ * 14, 14)])
        plsc.subcore_barrier()

        issue(2, 2)
        run_window(0, _WC)

        def wloop(w, _):
            p = w % 3
            wait_win(w, p)

            @pl.when(w + 2 < _WINS)
            def _():
                issue(w + 2, (w + 2) % 3)

            run_window(p, _WC)
            return 0

        lax.fori_loop(1, _WINS, wloop, 0)

        def mchunk(k_, _):
            rb = r0 + k_ * 16
            rowvec = lane_id + rb
            pltpu.sync_copy(acc.at[pl.ds(rb, 16)], spacc.at[rowvec], add=True)
            return 0

        lax.fori_loop(0, cnt, mchunk, 0)
        plsc.subcore_barrier()

        pltpu.sync_copy(spacc.at[pl.ds(s * 56, 56)],
                        out_hbm.at[c, pl.ds(s * 56, 56)])

    return k(idx1d, w1d)


def _combine_body(p_ref, o_ref):
    o_ref[...] = jnp.clip(jnp.sum(p_ref[...], axis=0), 0.0, 1.0)


def kernel(weights, ray_indices, num_rays):
    del num_rays
    idx = ray_indices.astype(jnp.int32)
    w = weights.astype(jnp.float32)

    partial = _sc_partial_sums(idx, w)

    out = pl.pallas_call(
        _combine_body,
        out_shape=jax.ShapeDtypeStruct((_SROWS, 128), jnp.float32),
    )(partial)
    return out.reshape(_SROWS * 128)[:_N_RAYS][:, None]


# speedup vs baseline: 108.8763x; 1.0005x over previous
import functools

import jax
import jax.numpy as jnp
from jax import lax
from jax.experimental import pallas as pl
from jax.experimental.pallas import tpu as pltpu
from jax.experimental.pallas import tpu_sc as plsc

_N_RAYS = 100000

_NC, _NS = 2, 16
_NW = _NC * _NS

_WC = 125
_CH = 16 * _WC
_WINS = 100
_TILE_SAMP = _WINS * _CH
_UNROLL = 5

_ACC = 102656
_DUMMY = _ACC - 1
_SROWS = 896


def _sc_partial_sums(idx1d, w1d):
    assert idx1d.shape[0] == _NW * _TILE_SAMP
    assert _WC % _UNROLL == 0

    mesh = plsc.VectorSubcoreMesh(core_axis_name="c", subcore_axis_name="s")

    @functools.partial(
        pl.kernel,
        out_type=jax.ShapeDtypeStruct((_NC, _SROWS, 128), jnp.float32),
        mesh=mesh,
        compiler_params=pltpu.CompilerParams(needs_layout_passes=False),
        scratch_types=[
            pltpu.VMEM((3 * _CH,), jnp.int32),
            pltpu.VMEM((3 * _CH,), jnp.float32),
            pltpu.VMEM((_ACC // 128, 128), jnp.float32),
            pltpu.VMEM((16,), jnp.int32),
            pltpu.VMEM_SHARED((_SROWS, 128), jnp.float32),
            pltpu.SemaphoreType.DMA((3,)),
        ],
    )
    def k(idx_hbm, w_hbm, out_hbm, ibuf, wbuf, acc, tailbuf, spacc, sem):
        c = lax.axis_index("c")
        s = lax.axis_index("s")
        tid = c * _NS + s
        tbase = tid * _TILE_SAMP

        lane_id = lax.iota(jnp.int32, 16)
        zeros16 = jnp.zeros((16,), jnp.float32)
        dummy16 = jnp.full((16,), _DUMMY, jnp.int32)

        def issue(w, p):
            base = tbase + w * _CH
            pltpu.make_async_copy(idx_hbm.at[pl.ds(base, _CH)],
                                  ibuf.at[pl.ds(p * _CH, _CH)],
                                  sem.at[p]).start()
            pltpu.make_async_copy(w_hbm.at[pl.ds(base, _CH)],
                                  wbuf.at[pl.ds(p * _CH, _CH)],
                                  sem.at[p]).start()

        def wait_win(w, p):
            base = tbase + w * _CH
            pltpu.make_async_copy(idx_hbm.at[pl.ds(base, _CH)],
                                  ibuf.at[pl.ds(p * _CH, _CH)],
                                  sem.at[p]).wait()
            pltpu.make_async_copy(w_hbm.at[pl.ds(base, _CH)],
                                  wbuf.at[pl.ds(p * _CH, _CH)],
                                  sem.at[p]).wait()

        def run_window(p, wc):
            lane_off = lane_id * wc + p * _CH

            def step_block(i, carry):
                cur, accv, base = carry
                t = i * _UNROLL
                loaded = []
                for u in range(_UNROLL):
                    ivec = lane_off + (t + u)
                    loaded.append((plsc.load_gather(ibuf, [ivec]),
                                   plsc.load_gather(wbuf, [ivec])))
                for idxv, wvv in loaded:
                    m = idxv != cur
                    plsc.addupdate_scatter(
                        acc,
                        [lax.shift_right_logical(cur, 7),
                         lax.bitwise_and(cur, 127)],
                        accv - base, mask=m)
                    base = jnp.where(m, accv, base)
                    accv = accv + wvv
                    cur = idxv
                return cur, accv, base

            cur, accv, base = lax.fori_loop(0, wc // _UNROLL, step_block,
                                            (dummy16, zeros16, zeros16))
            vals = accv - base
            for d in (1, 2, 4, 8):
                sh = jnp.maximum(lane_id - d, 0)
                sh_vals = vals.at[sh].get(mode="promise_in_bounds")
                sh_keys = cur.at[sh].get(mode="promise_in_bounds")
                take_m = (sh_keys == cur) & (lane_id >= d)
                vals = vals + jnp.where(take_m, sh_vals, 0.0)
            nxt = cur.at[jnp.minimum(lane_id + 1, 15)].get(
                mode="promise_in_bounds")
            is_last = (cur != nxt) | (lane_id == 15)
            plsc.addupdate_scatter(
                acc,
                [lax.shift_right_logical(cur, 7), lax.bitwise_and(cur, 127)],
                vals, mask=is_last)

        issue(0, 0)
        issue(1, 1)
        pltpu.sync_copy(idx_hbm.at[pl.ds(tbase + _TILE_SAMP - 16, 16)],
                        tailbuf)
        wait_win(0, 0)
        lo = ibuf[pl.ds(0, 16)][0]
        hi = tailbuf[...][15]
        r0 = lax.shift_right_logical(lo, 7)
        r1 = lax.shift_right_logical(hi, 7)
        cnt = lax.shift_right_logical(r1 - r0, 4) + 1

        def zchunk(k_, _):
            rb = r0 + k_ * 16
            for r in range(16):
                for u in range(8):
                    acc[rb + r, pl.ds(u * 16, 16)] = zeros16
            return 0

        lax.fori_loop(0, cnt, zchunk, 0)

        for k_ in range(4):
            pltpu.sync_copy(acc.at[pl.ds(r0, 14)],
                            spacc.at[pl.ds(s * 56 + k_ * 14, 14)])
        plsc.subcore_barrier()

        issue(2, 2)
        run_window(0, _WC)

        def wloop(w, _):
            p = w % 3
            wait_win(w, p)

            @pl.when(w + 2 < _WINS)
            def _():
                issue(w + 2, (w + 2) % 3)

            run_window(p, _WC)
            return 0

        lax.fori_loop(1, _WINS, wloop, 0)

        def mchunk(k_, _):
            rb = r0 + k_ * 16
            rowvec = lane_id + rb
            pltpu.sync_copy(acc.at[pl.ds(rb, 16)], spacc.at[rowvec], add=True)
            return 0

        lax.fori_loop(0, cnt, mchunk, 0)
        plsc.subcore_barrier()

        pltpu.sync_copy(spacc.at[pl.ds(s * 56, 56)],
                        out_hbm.at[c, pl.ds(s * 56, 56)])

    return k(idx1d, w1d)


def _combine_body(p_ref, o_ref):
    o_ref[...] = jnp.clip(jnp.sum(p_ref[...], axis=0), 0.0, 1.0)


def kernel(weights, ray_indices, num_rays):
    del num_rays
    idx = ray_indices.astype(jnp.int32)
    w = weights.astype(jnp.float32)

    partial = _sc_partial_sums(idx, w)

    out = pl.pallas_call(
        _combine_body,
        out_shape=jax.ShapeDtypeStruct((_SROWS, 128), jnp.float32),
    )(partial)
    return out.reshape(_SROWS * 128)[:_N_RAYS][:, None]
